# Initial kernel scaffold; baseline (speedup 1.0000x reference)
import jax
import jax.numpy as jnp
from jax.experimental import pallas as pl


def kernel(mem, val, store_index):
    raise NotImplementedError("write your pallas kernel here")


# speedup vs baseline: 2.2977x; 2.2977x over previous
import jax
import jax.numpy as jnp
from jax import lax
from jax.experimental import pallas as pl
from jax.experimental.pallas import tpu as pltpu


def _body(cap, size, blk, s_ref, mem_ref, valp_ref, out_ref):
    i = pl.program_id(0)
    b0 = i * blk
    s = jnp.remainder(b0 - s_ref[0], cap)
    sp = jnp.where(s >= cap - blk, s - cap, s)

    @pl.when(jnp.logical_or(sp >= size, sp <= -blk))
    def _copy():
        out_ref[...] = mem_ref[...]

    @pl.when(jnp.logical_and(sp < size, sp > -blk))
    def _overlay():
        start = jnp.clip(sp + blk, 0, size + blk)
        slab = valp_ref[pl.ds(start, blk), :]
        j = lax.broadcasted_iota(jnp.int32, (blk, mem_ref.shape[1]), 0)
        t = sp + j
        mask = jnp.logical_and(t >= 0, t < size)
        out_ref[...] = jnp.where(mask, slab, mem_ref[...])


def kernel(mem, val, store_index):
    cap, d = mem.shape
    size = min(val.shape[0], cap)
    blk = 8000
    assert cap % blk == 0 and blk <= size and blk <= cap - size
    grid = cap // blk

    s0 = jnp.remainder(jnp.asarray(store_index, jnp.int32), cap).reshape(1)
    val_pad = jnp.pad(val[:size], ((blk, blk), (0, 0)))

    import functools
    body = functools.partial(_body, cap, size, blk)
    return pl.pallas_call(
        body,
        out_shape=jax.ShapeDtypeStruct((cap, d), mem.dtype),
        grid=(grid,),
        in_specs=[
            pl.BlockSpec(memory_space=pltpu.SMEM),
            pl.BlockSpec((blk, d), lambda i: (i, 0)),
            pl.BlockSpec((size + 2 * blk, d), lambda i: (0, 0)),
        ],
        out_specs=pl.BlockSpec((blk, d), lambda i: (i, 0)),
        compiler_params=pltpu.CompilerParams(
            dimension_semantics=("arbitrary",),
        ),
    )(s0, mem, val_pad)
